# Initial kernel scaffold; baseline (speedup 1.0000x reference)
import jax
import jax.numpy as jnp
from jax.experimental import pallas as pl


def kernel(x, edge_index, edge_weight, W1, b1, W2, b2):
    raise NotImplementedError("write your pallas kernel here")


# speedup vs baseline: 5.9706x; 5.9706x over previous
import functools

import jax
import jax.numpy as jnp
from jax import lax
from jax.experimental import pallas as pl
from jax.experimental.pallas import tpu as pltpu
from jax.experimental.pallas import tpu_sc as plsc

N = 10000
E = 320000
D = 128
H = 64
C = 40
K = 10
ALPHA = 0.1

NC = 2
NS = 16
FH = H // NC
ROWS_T = 640
NPAD = NS * ROWS_T
CHUNK = 128
EPC = NS * CHUNK
E_PAD = ((E + EPC - 1) // EPC) * EPC
EDGES_T = E_PAD // NS
NCHUNK = EDGES_T // CHUNK
CINV = (1.0 - ALPHA) / float(K)


def _prop_body(y_hbm, row_hbm, col_hbm, w_hbm, z_hbm, h_hbm,
               dinv_v, hcur_v, acc_v, stage_v, gath_v,
               rowx_v, colx_v, wx_v, normx_v, deg1_v,
               deg_sp, dinv_sp, hnew_sp, sem):
    c = lax.axis_index("c")
    s = lax.axis_index("s")
    r0 = s * ROWS_T
    e0 = s * EDGES_T
    hoff = c * NPAD

    for i in range(ROWS_T // 16):
        deg1_v[pl.ds(i * 16, 16)] = jnp.full((16,), 1.0, jnp.float32)
    pltpu.sync_copy(deg1_v, deg_sp.at[pl.ds(r0, ROWS_T)])
    plsc.subcore_barrier()

    @pl.loop(0, NCHUNK)
    def _deg_chunk(j):
        base = e0 + j * CHUNK
        pltpu.sync_copy(row_hbm.at[pl.ds(base, CHUNK)], rowx_v)
        pltpu.sync_copy(w_hbm.at[pl.ds(base, CHUNK)], wx_v)
        pltpu.sync_copy(wx_v, deg_sp.at[rowx_v], add=True)

    plsc.subcore_barrier()

    pltpu.sync_copy(deg_sp.at[pl.ds(r0, ROWS_T)], deg1_v)
    for i in range(ROWS_T // 16):
        d16 = deg1_v[pl.ds(i * 16, 16)]
        bi = lax.bitcast_convert_type(d16, jnp.int32)
        bi = jnp.full((16,), 0x5F3759DF, jnp.int32) - (bi >> 1)
        r = lax.bitcast_convert_type(bi, jnp.float32)
        r = r * (1.5 - 0.5 * d16 * r * r)
        r = r * (1.5 - 0.5 * d16 * r * r)
        r = r * (1.5 - 0.5 * d16 * r * r)
        deg1_v[pl.ds(i * 16, 16)] = r
    pltpu.sync_copy(deg1_v, dinv_sp.at[pl.ds(r0, ROWS_T)])
    plsc.subcore_barrier()
    pltpu.sync_copy(dinv_sp, dinv_v)

    pltpu.sync_copy(y_hbm.at[pl.ds(hoff + r0, ROWS_T)], hcur_v)
    pltpu.sync_copy(hcur_v, h_hbm.at[pl.ds(hoff + r0, ROWS_T)])
    @pl.loop(0, ROWS_T)
    def _zacc(i):
        for f in range(FH // 16):
            acc_v[i, pl.ds(f * 16, 16)] = jnp.zeros((16,), jnp.float32)

    @pl.loop(0, K)
    def _round(_k):
        @pl.loop(0, ROWS_T // 16)
        def _selfinit(g):
            d16 = dinv_v[pl.ds(r0 + g * 16, 16)]
            s16 = d16 * d16
            for j in range(16):
                b = jnp.full((16,), s16[j], jnp.float32)
                for f in range(FH // 16):
                    stage_v[g * 16 + j, pl.ds(f * 16, 16)] = (
                        hcur_v[g * 16 + j, pl.ds(f * 16, 16)] * b)
        pltpu.sync_copy(stage_v, hnew_sp.at[pl.ds(r0, ROWS_T)])
        plsc.subcore_barrier()

        @pl.loop(0, NCHUNK)
        def _chunk(j):
            base = e0 + j * CHUNK
            pltpu.sync_copy(row_hbm.at[pl.ds(base, CHUNK)], rowx_v)
            pltpu.sync_copy(col_hbm.at[pl.ds(base, CHUNK)], colx_v)
            pltpu.sync_copy(w_hbm.at[pl.ds(base, CHUNK)], wx_v)
            for i in range(CHUNK // 16):
                r16 = rowx_v[pl.ds(i * 16, 16)]
                c16 = colx_v[pl.ds(i * 16, 16)]
                dr = plsc.load_gather(dinv_v, [r16])
                dc = plsc.load_gather(dinv_v, [c16])
                normx_v[pl.ds(i * 16, 16)] = dr * wx_v[pl.ds(i * 16, 16)] * dc
                colx_v[pl.ds(i * 16, 16)] = c16 + hoff
            pltpu.async_copy(h_hbm.at[colx_v], gath_v, sem).wait()

            for g in range(CHUNK // 16):
                n16 = normx_v[pl.ds(g * 16, 16)]
                for j in range(16):
                    b = jnp.full((16,), n16[j], jnp.float32)
                    for f in range(FH // 16):
                        gath_v[g * 16 + j, pl.ds(f * 16, 16)] = (
                            gath_v[g * 16 + j, pl.ds(f * 16, 16)] * b)

            pltpu.sync_copy(gath_v, hnew_sp.at[rowx_v], add=True)

        plsc.subcore_barrier()

        pltpu.sync_copy(hnew_sp.at[pl.ds(r0, ROWS_T)], stage_v)

        @pl.loop(0, ROWS_T)
        def _drain(i):
            for f in range(FH // 16):
                hn = stage_v[i, pl.ds(f * 16, 16)]
                acc_v[i, pl.ds(f * 16, 16)] = acc_v[i, pl.ds(f * 16, 16)] + hn
                hcur_v[i, pl.ds(f * 16, 16)] = hn
        pltpu.sync_copy(stage_v, h_hbm.at[pl.ds(hoff + r0, ROWS_T)])

    pltpu.sync_copy(y_hbm.at[pl.ds(hoff + r0, ROWS_T)], stage_v)

    @pl.loop(0, ROWS_T)
    def _mix(i):
        for f in range(FH // 16):
            acc_v[i, pl.ds(f * 16, 16)] = (
                CINV * acc_v[i, pl.ds(f * 16, 16)]
                + ALPHA * stage_v[i, pl.ds(f * 16, 16)])
    pltpu.sync_copy(acc_v, z_hbm.at[pl.ds(hoff + r0, ROWS_T)])


def _propagate(y_flat, rows_p, cols_p, w_p):
    mesh = plsc.VectorSubcoreMesh(core_axis_name="c", subcore_axis_name="s")
    f = pl.kernel(
        _prop_body,
        out_type=(
            jax.ShapeDtypeStruct((NC * NPAD, FH), jnp.float32),
            jax.ShapeDtypeStruct((NC * NPAD, FH), jnp.float32),
        ),
        mesh=mesh,
        compiler_params=pltpu.CompilerParams(
            needs_layout_passes=False, use_tc_tiling_on_sc=False),
        scratch_types=[
            pltpu.VMEM((NPAD,), jnp.float32),
            pltpu.VMEM((ROWS_T, FH), jnp.float32),
            pltpu.VMEM((ROWS_T, FH), jnp.float32),
            pltpu.VMEM((ROWS_T, FH), jnp.float32),
            pltpu.VMEM((CHUNK, FH), jnp.float32),
            pltpu.VMEM((CHUNK,), jnp.int32),
            pltpu.VMEM((CHUNK,), jnp.int32),
            pltpu.VMEM((CHUNK,), jnp.float32),
            pltpu.VMEM((CHUNK,), jnp.float32),
            pltpu.VMEM((ROWS_T,), jnp.float32),
            pltpu.VMEM_SHARED((NPAD,), jnp.float32),
            pltpu.VMEM_SHARED((NPAD,), jnp.float32),
            pltpu.VMEM_SHARED((NPAD, FH), jnp.float32),
            pltpu.SemaphoreType.DMA,
        ],
    )
    z, _h = f(y_flat, rows_p, cols_p, w_p)
    return z


def _mm1_body(x_ref, w_ref, o_ref):
    o_ref[...] = jax.lax.dot(x_ref[...], w_ref[...],
                             preferred_element_type=jnp.float32)


def _mm2_body(z_ref, b1_ref, w2_ref, b2_ref, o_ref):
    a = jax.nn.relu(z_ref[...] + b1_ref[...])
    o_ref[...] = jax.lax.dot(a, w2_ref[...],
                             preferred_element_type=jnp.float32) + b2_ref[...]


def kernel(x, edge_index, edge_weight, W1, b1, W2, b2):
    y = pl.pallas_call(
        _mm1_body,
        grid=(10,),
        in_specs=[
            pl.BlockSpec((N // 10, D), lambda i: (i, 0)),
            pl.BlockSpec((D, H), lambda i: (0, 0)),
        ],
        out_specs=pl.BlockSpec((N // 10, H), lambda i: (i, 0)),
        out_shape=jax.ShapeDtypeStruct((N, H), jnp.float32),
    )(x, W1)

    ypad = jnp.zeros((NPAD, H), jnp.float32).at[:N].set(y)
    y_flat = jnp.concatenate([ypad[:, :FH], ypad[:, FH:]], axis=0)
    npad_e = E_PAD - E
    spread = (jnp.arange(npad_e, dtype=jnp.int32) * 37) % N
    rows_p = jnp.concatenate([edge_index[0], spread])
    cols_p = jnp.concatenate([edge_index[1], spread])
    w_p = jnp.concatenate([edge_weight, jnp.zeros((npad_e,), jnp.float32)])

    z_flat = _propagate(y_flat, rows_p, cols_p, w_p)
    z = jnp.concatenate([z_flat[:N], z_flat[NPAD:NPAD + N]], axis=1)

    out = pl.pallas_call(
        _mm2_body,
        grid=(10,),
        in_specs=[
            pl.BlockSpec((N // 10, H), lambda i: (i, 0)),
            pl.BlockSpec((H,), lambda i: (0,)),
            pl.BlockSpec((H, C), lambda i: (0, 0)),
            pl.BlockSpec((C,), lambda i: (0,)),
        ],
        out_specs=pl.BlockSpec((N // 10, C), lambda i: (i, 0)),
        out_shape=jax.ShapeDtypeStruct((N, C), jnp.float32),
    )(z, b1, W2, b2)
    return out


# speedup vs baseline: 16.2413x; 2.7202x over previous
import functools

import jax
import jax.numpy as jnp
from jax import lax
from jax.experimental import pallas as pl
from jax.experimental.pallas import tpu as pltpu
from jax.experimental.pallas import tpu_sc as plsc

N = 10000
E = 320000
D = 128
H = 64
C = 40
K = 10
ALPHA = 0.1

NC = 2
NS = 16
FH = H // NC
ROWS_T = 640
NPAD = NS * ROWS_T
CHUNK = 128
NCH_T = 160
NSUPER = 10
E_PAD = NS * NCH_T * CHUNK
NCHUNKS = E_PAD // CHUNK
CINV = (1.0 - ALPHA) / float(K)


def _prop_body(y_hbm, row_hbm, col_hbm, w_hbm, z_hbm, h_hbm,
               dinv_v, hcur_v, acc_v, norm2_v, cadj_v,
               rowx2_v, cstage_v, wx2_v, gA, gB, deg1_v,
               deg_sp, dinv_sp, hnew_sp, sem_g0, sem_g1):
    c = lax.axis_index("c")
    s = lax.axis_index("s")
    r0 = s * ROWS_T
    ch0 = s * NCH_T
    hoff = c * NPAD

    for i in range(ROWS_T // 16):
        deg1_v[pl.ds(i * 16, 16)] = jnp.full((16,), 1.0, jnp.float32)
    pltpu.sync_copy(deg1_v, deg_sp.at[pl.ds(r0, ROWS_T)])
    plsc.subcore_barrier()

    @pl.loop(0, NSUPER)
    def _deg(j):
        pltpu.sync_copy(row_hbm.at[pl.ds(ch0 + j * 16, 16)], rowx2_v)
        pltpu.sync_copy(w_hbm.at[pl.ds(ch0 + j * 16, 16)], wx2_v)
        for k in range(16):
            pltpu.sync_copy(wx2_v.at[k], deg_sp.at[rowx2_v.at[k]], add=True)

    plsc.subcore_barrier()

    pltpu.sync_copy(deg_sp.at[pl.ds(r0, ROWS_T)], deg1_v)
    for i in range(ROWS_T // 16):
        d16 = deg1_v[pl.ds(i * 16, 16)]
        bi = lax.bitcast_convert_type(d16, jnp.int32)
        bi = jnp.full((16,), 0x5F3759DF, jnp.int32) - (bi >> 1)
        r = lax.bitcast_convert_type(bi, jnp.float32)
        r = r * (1.5 - 0.5 * d16 * r * r)
        r = r * (1.5 - 0.5 * d16 * r * r)
        r = r * (1.5 - 0.5 * d16 * r * r)
        deg1_v[pl.ds(i * 16, 16)] = r
    pltpu.sync_copy(deg1_v, dinv_sp.at[pl.ds(r0, ROWS_T)])
    plsc.subcore_barrier()
    pltpu.sync_copy(dinv_sp, dinv_v)

    @pl.loop(0, NSUPER)
    def _pre(j):
        pltpu.sync_copy(row_hbm.at[pl.ds(ch0 + j * 16, 16)], rowx2_v)
        pltpu.sync_copy(col_hbm.at[pl.ds(ch0 + j * 16, 16)], cstage_v)
        pltpu.sync_copy(w_hbm.at[pl.ds(ch0 + j * 16, 16)], wx2_v)

        @pl.loop(0, 16)
        def _prechunk(k):
            t = j * 16 + k
            for g in range(8):
                r16 = rowx2_v[k, pl.ds(g * 16, 16)]
                c16 = cstage_v[k, pl.ds(g * 16, 16)]
                dr = plsc.load_gather(dinv_v, [r16])
                dc = plsc.load_gather(dinv_v, [c16])
                norm2_v[t, pl.ds(g * 16, 16)] = (
                    dr * wx2_v[k, pl.ds(g * 16, 16)] * dc)
                cadj_v[t, pl.ds(g * 16, 16)] = c16 + hoff

    pltpu.sync_copy(y_hbm.at[pl.ds(hoff + r0, ROWS_T)], hcur_v)
    pltpu.sync_copy(hcur_v, h_hbm.at[pl.ds(hoff + r0, ROWS_T)])

    @pl.loop(0, ROWS_T)
    def _zacc(i):
        for f in range(FH // 16):
            acc_v[i, pl.ds(f * 16, 16)] = jnp.zeros((16,), jnp.float32)

    @pl.loop(0, K)
    def _round(_k):
        @pl.loop(0, ROWS_T // 128)
        def _si(blk):
            @pl.loop(0, 8)
            def _sig(g):
                base = blk * 128 + g * 16
                d16 = dinv_v[pl.ds(r0 + base, 16)]
                s16 = d16 * d16
                for jj in range(16):
                    b = jnp.full((16,), s16[jj], jnp.float32)
                    for f in range(FH // 16):
                        gA[g * 16 + jj, pl.ds(f * 16, 16)] = (
                            hcur_v[base + jj, pl.ds(f * 16, 16)] * b)
            pltpu.sync_copy(gA, hnew_sp.at[pl.ds(r0 + blk * 128, 128)])

        plsc.subcore_barrier()

        pltpu.async_copy(h_hbm.at[cadj_v.at[0]], gA, sem_g0)

        @pl.loop(0, NSUPER)
        def _super(j):
            pltpu.sync_copy(row_hbm.at[pl.ds(ch0 + j * 16, 16)], rowx2_v)
            for k in range(16):
                t = j * 16 + k
                if k % 2 == 0:
                    gb, sem, go, semo = gA, sem_g0, gB, sem_g1
                else:
                    gb, sem, go, semo = gB, sem_g1, gA, sem_g0
                pltpu.make_async_copy(h_hbm.at[pl.ds(0, CHUNK)], gb, sem).wait()

                @pl.when(t < NCH_T - 1)
                def _pf():
                    pltpu.async_copy(h_hbm.at[cadj_v.at[t + 1]], go, semo)

                @pl.loop(0, 8)
                def _sc(g):
                    n16 = norm2_v[t, pl.ds(g * 16, 16)]
                    for jj in range(16):
                        b = jnp.full((16,), n16[jj], jnp.float32)
                        for f in range(FH // 16):
                            gb[g * 16 + jj, pl.ds(f * 16, 16)] = (
                                gb[g * 16 + jj, pl.ds(f * 16, 16)] * b)

                pltpu.sync_copy(gb, hnew_sp.at[rowx2_v.at[k]], add=True)

        plsc.subcore_barrier()

        @pl.loop(0, ROWS_T // 128)
        def _dr(blk):
            pltpu.sync_copy(hnew_sp.at[pl.ds(r0 + blk * 128, 128)], gB)

            @pl.loop(0, 8)
            def _drg(g):
                base = blk * 128 + g * 16
                for jj in range(16):
                    for f in range(FH // 16):
                        hn = gB[g * 16 + jj, pl.ds(f * 16, 16)]
                        acc_v[base + jj, pl.ds(f * 16, 16)] = (
                            acc_v[base + jj, pl.ds(f * 16, 16)] + hn)
                        hcur_v[base + jj, pl.ds(f * 16, 16)] = hn

        pltpu.sync_copy(hcur_v, h_hbm.at[pl.ds(hoff + r0, ROWS_T)])

    @pl.loop(0, ROWS_T // 128)
    def _fm(blk):
        pltpu.sync_copy(y_hbm.at[pl.ds(hoff + r0 + blk * 128, CHUNK)], gA)

        @pl.loop(0, 8)
        def _fmg(g):
            base = blk * 128 + g * 16
            for jj in range(16):
                for f in range(FH // 16):
                    acc_v[base + jj, pl.ds(f * 16, 16)] = (
                        CINV * acc_v[base + jj, pl.ds(f * 16, 16)]
                        + ALPHA * gA[g * 16 + jj, pl.ds(f * 16, 16)])

    pltpu.sync_copy(acc_v, z_hbm.at[pl.ds(hoff + r0, ROWS_T)])


def _propagate(y_flat, rows_p, cols_p, w_p):
    mesh = plsc.VectorSubcoreMesh(core_axis_name="c", subcore_axis_name="s")
    f = pl.kernel(
        _prop_body,
        out_type=(
            jax.ShapeDtypeStruct((NC * NPAD, FH), jnp.float32),
            jax.ShapeDtypeStruct((NC * NPAD, FH), jnp.float32),
        ),
        mesh=mesh,
        compiler_params=pltpu.CompilerParams(
            needs_layout_passes=False, use_tc_tiling_on_sc=False),
        scratch_types=[
            pltpu.VMEM((NPAD,), jnp.float32),
            pltpu.VMEM((ROWS_T, FH), jnp.float32),
            pltpu.VMEM((ROWS_T, FH), jnp.float32),
            pltpu.VMEM((NCH_T, CHUNK), jnp.float32),
            pltpu.VMEM((NCH_T, CHUNK), jnp.int32),
            pltpu.VMEM((16, CHUNK), jnp.int32),
            pltpu.VMEM((16, CHUNK), jnp.int32),
            pltpu.VMEM((16, CHUNK), jnp.float32),
            pltpu.VMEM((CHUNK, FH), jnp.float32),
            pltpu.VMEM((CHUNK, FH), jnp.float32),
            pltpu.VMEM((ROWS_T,), jnp.float32),
            pltpu.VMEM_SHARED((NPAD,), jnp.float32),
            pltpu.VMEM_SHARED((NPAD,), jnp.float32),
            pltpu.VMEM_SHARED((NPAD, FH), jnp.float32),
            pltpu.SemaphoreType.DMA,
            pltpu.SemaphoreType.DMA,
        ],
    )
    z, _h = f(y_flat, rows_p, cols_p, w_p)
    return z


def _mm1_body(x_ref, w_ref, o_ref):
    o_ref[...] = jax.lax.dot(x_ref[...], w_ref[...],
                             preferred_element_type=jnp.float32)


def _mm2_body(z_ref, b1_ref, w2_ref, b2_ref, o_ref):
    a = jax.nn.relu(z_ref[...] + b1_ref[...])
    o_ref[...] = jax.lax.dot(a, w2_ref[...],
                             preferred_element_type=jnp.float32) + b2_ref[...]


def kernel(x, edge_index, edge_weight, W1, b1, W2, b2):
    y = pl.pallas_call(
        _mm1_body,
        grid=(10,),
        in_specs=[
            pl.BlockSpec((N // 10, D), lambda i: (i, 0)),
            pl.BlockSpec((D, H), lambda i: (0, 0)),
        ],
        out_specs=pl.BlockSpec((N // 10, H), lambda i: (i, 0)),
        out_shape=jax.ShapeDtypeStruct((N, H), jnp.float32),
    )(x, W1)

    ypad = jnp.zeros((NPAD, H), jnp.float32).at[:N].set(y)
    y_flat = jnp.concatenate([ypad[:, :FH], ypad[:, FH:]], axis=0)
    npad_e = E_PAD - E
    spread = (jnp.arange(npad_e, dtype=jnp.int32) * 37) % N
    rows_p = jnp.concatenate([edge_index[0], spread]).reshape(NCHUNKS, CHUNK)
    cols_p = jnp.concatenate([edge_index[1], spread]).reshape(NCHUNKS, CHUNK)
    w_p = jnp.concatenate(
        [edge_weight, jnp.zeros((npad_e,), jnp.float32)]).reshape(
            NCHUNKS, CHUNK)

    z_flat = _propagate(y_flat, rows_p, cols_p, w_p)
    z = jnp.concatenate([z_flat[:N], z_flat[NPAD:NPAD + N]], axis=1)

    out = pl.pallas_call(
        _mm2_body,
        grid=(10,),
        in_specs=[
            pl.BlockSpec((N // 10, H), lambda i: (i, 0)),
            pl.BlockSpec((H,), lambda i: (0,)),
            pl.BlockSpec((H, C), lambda i: (0, 0)),
            pl.BlockSpec((C,), lambda i: (0,)),
        ],
        out_specs=pl.BlockSpec((N // 10, C), lambda i: (i, 0)),
        out_shape=jax.ShapeDtypeStruct((N, C), jnp.float32),
    )(z, b1, W2, b2)
    return out


# speedup vs baseline: 23.7599x; 1.4629x over previous
import functools

import jax
import jax.numpy as jnp
from jax import lax
from jax.experimental import pallas as pl
from jax.experimental.pallas import tpu as pltpu
from jax.experimental.pallas import tpu_sc as plsc

N = 10000
E = 320000
D = 128
H = 64
C = 40
K = 10
ALPHA = 0.1

NC = 2
NS = 16
FH = H // NC
ROWS_T = 640
NPAD = NS * ROWS_T
CHUNK = 128
SUPER = 15
NSUPER = 11
NCH_T = SUPER * NSUPER
E_PAD = NS * NCH_T * CHUNK
NCHUNKS = E_PAD // CHUNK
CINV = (1.0 - ALPHA) / float(K)


def _prop_body(y_hbm, row_hbm, col_hbm, w_hbm, z_hbm, h_hbm,
               dinv_v, acc_v, norm2_v, cadj_v, rowx_v, wx2_v,
               g0, g1, g2, deg1_v,
               deg_sp, dinv_sp, hnew_sp,
               sg0, sg1, sg2, ss0, ss1, ss2):
    c = lax.axis_index("c")
    s = lax.axis_index("s")
    r0 = s * ROWS_T
    ch0 = s * NCH_T
    hoff = c * NPAD
    ring = (g0, g1, g2)
    sgs = (sg0, sg1, sg2)
    sss = (ss0, ss1, ss2)

    for i in range(ROWS_T // 16):
        deg1_v[pl.ds(i * 16, 16)] = jnp.full((16,), 1.0, jnp.float32)
    pltpu.sync_copy(deg1_v, deg_sp.at[pl.ds(r0, ROWS_T)])
    plsc.subcore_barrier()

    @pl.loop(0, NSUPER)
    def _deg(j):
        pltpu.sync_copy(row_hbm.at[pl.ds(ch0 + j * SUPER, SUPER)],
                        rowx_v.at[pl.ds(j * SUPER, SUPER)])
        pltpu.sync_copy(w_hbm.at[pl.ds(ch0 + j * SUPER, SUPER)], wx2_v)
        for k in range(SUPER):
            pltpu.sync_copy(wx2_v.at[k], deg_sp.at[rowx_v.at[j * SUPER + k]],
                            add=True)

    plsc.subcore_barrier()

    pltpu.sync_copy(deg_sp.at[pl.ds(r0, ROWS_T)], deg1_v)
    for i in range(ROWS_T // 16):
        d16 = deg1_v[pl.ds(i * 16, 16)]
        bi = lax.bitcast_convert_type(d16, jnp.int32)
        bi = jnp.full((16,), 0x5F3759DF, jnp.int32) - (bi >> 1)
        r = lax.bitcast_convert_type(bi, jnp.float32)
        r = r * (1.5 - 0.5 * d16 * r * r)
        r = r * (1.5 - 0.5 * d16 * r * r)
        r = r * (1.5 - 0.5 * d16 * r * r)
        deg1_v[pl.ds(i * 16, 16)] = r
    pltpu.sync_copy(deg1_v, dinv_sp.at[pl.ds(r0, ROWS_T)])
    plsc.subcore_barrier()
    pltpu.sync_copy(dinv_sp, dinv_v)

    @pl.loop(0, NSUPER)
    def _pre(j):
        pltpu.sync_copy(col_hbm.at[pl.ds(ch0 + j * SUPER, SUPER)],
                        cadj_v.at[pl.ds(j * SUPER, SUPER)])
        pltpu.sync_copy(w_hbm.at[pl.ds(ch0 + j * SUPER, SUPER)], wx2_v)

        @pl.loop(0, SUPER)
        def _prechunk(k):
            t = j * SUPER + k
            for g in range(8):
                r16 = rowx_v[t, pl.ds(g * 16, 16)]
                c16 = cadj_v[t, pl.ds(g * 16, 16)]
                dr = plsc.load_gather(dinv_v, [r16])
                dc = plsc.load_gather(dinv_v, [c16])
                norm2_v[t, pl.ds(g * 16, 16)] = (
                    dr * wx2_v[k, pl.ds(g * 16, 16)] * dc)
                cadj_v[t, pl.ds(g * 16, 16)] = c16 + hoff

    @pl.loop(0, ROWS_T)
    def _zacc(i):
        for f in range(FH // 16):
            acc_v[i, pl.ds(f * 16, 16)] = jnp.zeros((16,), jnp.float32)

    @pl.loop(0, CHUNK)
    def _zg1(i):
        for f in range(FH // 16):
            g1[i, pl.ds(f * 16, 16)] = jnp.zeros((16,), jnp.float32)

    @pl.loop(0, ROWS_T // CHUNK)
    def _inith(blk):
        pltpu.sync_copy(y_hbm.at[pl.ds(hoff + r0 + blk * CHUNK, CHUNK)], g0)
        pltpu.sync_copy(g0, h_hbm.at[pl.ds(hoff + r0 + blk * CHUNK, CHUNK)])
        pltpu.sync_copy(g1, hnew_sp.at[pl.ds(r0 + blk * CHUNK, CHUNK)])

    plsc.subcore_barrier()

    @pl.loop(0, K)
    def _round(_k):
        pltpu.async_copy(h_hbm.at[cadj_v.at[0]], g0, sg0)
        pltpu.async_copy(h_hbm.at[cadj_v.at[1]], g1, sg1)

        @pl.loop(0, NSUPER)
        def _super(j):
            for k in range(SUPER):
                t = j * SUPER + k
                b = k % 3
                gb, semg, sems = ring[b], sgs[b], sss[b]
                b2 = (k + 2) % 3
                gn, semgn, semsn = ring[b2], sgs[b2], sss[b2]

                pltpu.make_async_copy(
                    h_hbm.at[pl.ds(0, CHUNK)], gb, semg).wait()

                @pl.when(jnp.logical_and(t >= 1, t + 2 < NCH_T))
                def _ws():
                    pltpu.make_async_copy(
                        gn, hnew_sp.at[pl.ds(0, CHUNK)], semsn).wait()

                @pl.when(t + 2 < NCH_T)
                def _pf():
                    pltpu.async_copy(h_hbm.at[cadj_v.at[t + 2]], gn, semgn)

                @pl.loop(0, 8)
                def _sc(g):
                    n16 = norm2_v[t, pl.ds(g * 16, 16)]
                    for jj in range(16):
                        bcast = jnp.full((16,), n16[jj], jnp.float32)
                        for f in range(FH // 16):
                            gb[g * 16 + jj, pl.ds(f * 16, 16)] = (
                                gb[g * 16 + jj, pl.ds(f * 16, 16)] * bcast)

                pltpu.async_copy(gb, hnew_sp.at[rowx_v.at[t]], sems, add=True)

        for b in ((NCH_T - 3) % 3, (NCH_T - 2) % 3, (NCH_T - 1) % 3):
            pltpu.make_async_copy(
                ring[b], hnew_sp.at[pl.ds(0, CHUNK)], sss[b]).wait()

        plsc.subcore_barrier()

        @pl.loop(0, CHUNK)
        def _zg1r(i):
            for f in range(FH // 16):
                g1[i, pl.ds(f * 16, 16)] = jnp.zeros((16,), jnp.float32)

        @pl.loop(0, ROWS_T // CHUNK)
        def _dr(blk):
            pltpu.sync_copy(hnew_sp.at[pl.ds(r0 + blk * CHUNK, CHUNK)], g0)
            pltpu.sync_copy(
                h_hbm.at[pl.ds(hoff + r0 + blk * CHUNK, CHUNK)], g2)

            @pl.loop(0, 8)
            def _drg(g):
                base = blk * CHUNK + g * 16
                d16 = dinv_v[pl.ds(r0 + base, 16)]
                s16 = d16 * d16
                for jj in range(16):
                    bcast = jnp.full((16,), s16[jj], jnp.float32)
                    for f in range(FH // 16):
                        hn = (g0[g * 16 + jj, pl.ds(f * 16, 16)]
                              + g2[g * 16 + jj, pl.ds(f * 16, 16)] * bcast)
                        g2[g * 16 + jj, pl.ds(f * 16, 16)] = hn
                        acc_v[base + jj, pl.ds(f * 16, 16)] = (
                            acc_v[base + jj, pl.ds(f * 16, 16)] + hn)

            pltpu.sync_copy(g1, hnew_sp.at[pl.ds(r0 + blk * CHUNK, CHUNK)])
            pltpu.sync_copy(
                g2, h_hbm.at[pl.ds(hoff + r0 + blk * CHUNK, CHUNK)])

        plsc.subcore_barrier()

    @pl.loop(0, ROWS_T // CHUNK)
    def _fm(blk):
        pltpu.sync_copy(y_hbm.at[pl.ds(hoff + r0 + blk * CHUNK, CHUNK)], g0)

        @pl.loop(0, 8)
        def _fmg(g):
            base = blk * CHUNK + g * 16
            for jj in range(16):
                for f in range(FH // 16):
                    acc_v[base + jj, pl.ds(f * 16, 16)] = (
                        CINV * acc_v[base + jj, pl.ds(f * 16, 16)]
                        + ALPHA * g0[g * 16 + jj, pl.ds(f * 16, 16)])

    pltpu.sync_copy(acc_v, z_hbm.at[pl.ds(hoff + r0, ROWS_T)])


def _propagate(y_flat, rows_p, cols_p, w_p):
    mesh = plsc.VectorSubcoreMesh(core_axis_name="c", subcore_axis_name="s")
    f = pl.kernel(
        _prop_body,
        out_type=(
            jax.ShapeDtypeStruct((NC * NPAD, FH), jnp.float32),
            jax.ShapeDtypeStruct((NC * NPAD, FH), jnp.float32),
        ),
        mesh=mesh,
        compiler_params=pltpu.CompilerParams(
            needs_layout_passes=False, use_tc_tiling_on_sc=False),
        scratch_types=[
            pltpu.VMEM((NPAD,), jnp.float32),
            pltpu.VMEM((ROWS_T, FH), jnp.float32),
            pltpu.VMEM((NCH_T, CHUNK), jnp.float32),
            pltpu.VMEM((NCH_T, CHUNK), jnp.int32),
            pltpu.VMEM((NCH_T, CHUNK), jnp.int32),
            pltpu.VMEM((SUPER, CHUNK), jnp.float32),
            pltpu.VMEM((CHUNK, FH), jnp.float32),
            pltpu.VMEM((CHUNK, FH), jnp.float32),
            pltpu.VMEM((CHUNK, FH), jnp.float32),
            pltpu.VMEM((ROWS_T,), jnp.float32),
            pltpu.VMEM_SHARED((NPAD,), jnp.float32),
            pltpu.VMEM_SHARED((NPAD,), jnp.float32),
            pltpu.VMEM_SHARED((NPAD, FH), jnp.float32),
            pltpu.SemaphoreType.DMA,
            pltpu.SemaphoreType.DMA,
            pltpu.SemaphoreType.DMA,
            pltpu.SemaphoreType.DMA,
            pltpu.SemaphoreType.DMA,
            pltpu.SemaphoreType.DMA,
        ],
    )
    z, _h = f(y_flat, rows_p, cols_p, w_p)
    return z


def _mm1_body(x_ref, w_ref, o_ref):
    o_ref[...] = jax.lax.dot(x_ref[...], w_ref[...],
                             preferred_element_type=jnp.float32)


def _mm2_body(z_ref, b1_ref, w2_ref, b2_ref, o_ref):
    a = jax.nn.relu(z_ref[...] + b1_ref[...])
    o_ref[...] = jax.lax.dot(a, w2_ref[...],
                             preferred_element_type=jnp.float32) + b2_ref[...]


def kernel(x, edge_index, edge_weight, W1, b1, W2, b2):
    y = pl.pallas_call(
        _mm1_body,
        grid=(10,),
        in_specs=[
            pl.BlockSpec((N // 10, D), lambda i: (i, 0)),
            pl.BlockSpec((D, H), lambda i: (0, 0)),
        ],
        out_specs=pl.BlockSpec((N // 10, H), lambda i: (i, 0)),
        out_shape=jax.ShapeDtypeStruct((N, H), jnp.float32),
    )(x, W1)

    ypad = jnp.zeros((NPAD, H), jnp.float32).at[:N].set(y)
    y_flat = jnp.concatenate([ypad[:, :FH], ypad[:, FH:]], axis=0)
    npad_e = E_PAD - E
    spread = (jnp.arange(npad_e, dtype=jnp.int32) * 37) % N
    rows_p = jnp.concatenate([edge_index[0], spread]).reshape(NCHUNKS, CHUNK)
    cols_p = jnp.concatenate([edge_index[1], spread]).reshape(NCHUNKS, CHUNK)
    w_p = jnp.concatenate(
        [edge_weight, jnp.zeros((npad_e,), jnp.float32)]).reshape(
            NCHUNKS, CHUNK)

    z_flat = _propagate(y_flat, rows_p, cols_p, w_p)
    z = jnp.concatenate([z_flat[:N], z_flat[NPAD:NPAD + N]], axis=1)

    out = pl.pallas_call(
        _mm2_body,
        grid=(10,),
        in_specs=[
            pl.BlockSpec((N // 10, H), lambda i: (i, 0)),
            pl.BlockSpec((H,), lambda i: (0,)),
            pl.BlockSpec((H, C), lambda i: (0, 0)),
            pl.BlockSpec((C,), lambda i: (0,)),
        ],
        out_specs=pl.BlockSpec((N // 10, C), lambda i: (i, 0)),
        out_shape=jax.ShapeDtypeStruct((N, C), jnp.float32),
    )(z, b1, W2, b2)
    return out
